# baseline (speedup 1.0000x reference)
import functools

import jax
import jax.numpy as jnp
from jax import lax
from jax.experimental import pallas as pl
from jax.experimental.pallas import tpu as pltpu
from jax.experimental.pallas import tpu_sc as plsc

_LANES = 16


def _sc_gather_sum(ids, table):
    B, L = ids.shape
    _, H2 = table.shape
    H = 2 * H2
    NC, NS = 2, 16
    NW = NC * NS
    RPW = B // NW
    CH = H2 // _LANES

    mesh = plsc.VectorSubcoreMesh(core_axis_name="c", subcore_axis_name="s")

    @functools.partial(
        pl.kernel,
        out_type=jax.ShapeDtypeStruct((B, H), jnp.float32),
        mesh=mesh,
        scratch_types=[
            pltpu.VMEM((RPW, L), jnp.int32),
            pltpu.VMEM((L, H2), jnp.int32),
            pltpu.VMEM((L, H2), jnp.int32),
            pltpu.VMEM((RPW, H), jnp.float32),
            pltpu.SemaphoreType.DMA,
            pltpu.SemaphoreType.DMA,
        ],
    )
    def sc_kernel(ids_hbm, table_hbm, out_hbm, idx_v, buf0, buf1, out_v, sem0, sem1):
        wid = lax.axis_index("s") * NC + lax.axis_index("c")
        base = wid * RPW
        pltpu.sync_copy(ids_hbm.at[pl.ds(base, RPW)], idx_v)

        def reduce_into(buf, r):
            def body(i, accs):
                new = []
                for j in range(CH):
                    w = buf[i, pl.ds(_LANES * j, _LANES)]
                    even = lax.bitcast_convert_type(
                        lax.shift_left(w, 16), jnp.float32)
                    odd = lax.bitcast_convert_type(
                        lax.bitwise_and(w, jnp.int32(-65536)), jnp.float32)
                    new.append(accs[2 * j] + even)
                    new.append(accs[2 * j + 1] + odd)
                return tuple(new)
            accs = lax.fori_loop(
                0, L, body,
                tuple(jnp.zeros((_LANES,), jnp.float32) for _ in range(2 * CH)))
            for j in range(2 * CH):
                out_v[r, pl.ds(_LANES * j, _LANES)] = accs[j]

        pltpu.async_copy(table_hbm.at[idx_v.at[0]], buf0, sem0)

        def loop_body(g, carry):
            r0 = 2 * g
            pltpu.async_copy(table_hbm.at[idx_v.at[r0 + 1]], buf1, sem1)
            pltpu.make_async_copy(table_hbm.at[idx_v.at[r0]], buf0, sem0).wait()
            reduce_into(buf0, r0)

            @pl.when(r0 + 2 < RPW)
            def _():
                pltpu.async_copy(table_hbm.at[idx_v.at[r0 + 2]], buf0, sem0)

            pltpu.make_async_copy(table_hbm.at[idx_v.at[r0 + 1]], buf1, sem1).wait()
            reduce_into(buf1, r0 + 1)
            return carry

        lax.fori_loop(0, RPW // 2, loop_body, 0)
        pltpu.sync_copy(out_v, out_hbm.at[pl.ds(base, RPW)])

    return sc_kernel(ids, table)


def _tc_mlp(sums, pos, W1, b1, W2, b2, inv_l):
    B, H = sums.shape
    L = pos.shape[0]
    BM = 512

    def mlp_kernel(s_ref, pos_ref, w1_ref, b1_ref, w2_ref, b2_ref, o_ref):
        pos_mean = jnp.mean(pos_ref[...], axis=0, keepdims=True)
        x = s_ref[...] * inv_l + pos_mean
        h = lax.dot_general(x, w1_ref[...], (((1,), (1,)), ((), ())),
                            preferred_element_type=jnp.float32)
        h = jnp.maximum(h + b1_ref[...], 0.0)
        o = lax.dot_general(h, w2_ref[...], (((1,), (1,)), ((), ())),
                            preferred_element_type=jnp.float32)
        o_ref[...] = o + b2_ref[...]

    return pl.pallas_call(
        mlp_kernel,
        grid=(B // BM,),
        in_specs=[
            pl.BlockSpec((BM, H), lambda i: (i, 0)),
            pl.BlockSpec((L, H), lambda i: (0, 0)),
            pl.BlockSpec(W1.shape, lambda i: (0, 0)),
            pl.BlockSpec((1, H), lambda i: (0, 0)),
            pl.BlockSpec(W2.shape, lambda i: (0, 0)),
            pl.BlockSpec((1, H), lambda i: (0, 0)),
        ],
        out_specs=pl.BlockSpec((BM, W2.shape[0]), lambda i: (i, 0)),
        out_shape=jax.ShapeDtypeStruct((B, W2.shape[0]), jnp.float32),
    )(sums, pos, W1, b1, W2, b2)


def _evenodd(x):
    lead, H = x.shape[:-1], x.shape[-1]
    return x.reshape(*lead, H // 32, 16, 2).swapaxes(-1, -2).reshape(*lead, H)


def kernel(input_ids, emb_table, pos_table, W1, b1, W2, b2):
    ids = input_ids.astype(jnp.int32)
    L = ids.shape[1]
    V, H = emb_table.shape
    table_i32 = jax.lax.bitcast_convert_type(
        emb_table.astype(jnp.bfloat16).reshape(V, H // 2, 2), jnp.int32)
    sums = _sc_gather_sum(ids, table_i32)
    return _tc_mlp(sums, _evenodd(pos_table[:L]), _evenodd(W1),
                   b1.reshape(1, -1), W2, b2.reshape(1, -1), 1.0 / L)


# speedup vs baseline: 3.3562x; 3.3562x over previous
import functools

import jax
import jax.numpy as jnp
from jax import lax
from jax.experimental import pallas as pl
from jax.experimental.pallas import tpu as pltpu
from jax.experimental.pallas import tpu_sc as plsc

_LANES = 16


def _sc_gather_sum(ids, table):
    B, L = ids.shape
    _, H = table.shape
    NC, NS = 2, 16
    NW = NC * NS
    RPW = B // NW
    CH = H // _LANES

    HR = L // 2
    mesh = plsc.VectorSubcoreMesh(core_axis_name="c", subcore_axis_name="s")

    @functools.partial(
        pl.kernel,
        out_type=jax.ShapeDtypeStruct((B, H), jnp.float32),
        mesh=mesh,
        scratch_types=[
            pltpu.VMEM((RPW, L), jnp.int32),
            pltpu.VMEM((HR, H), jnp.float32),
            pltpu.VMEM((HR, H), jnp.float32),
            pltpu.VMEM((HR, H), jnp.float32),
            pltpu.VMEM((HR, H), jnp.float32),
            pltpu.VMEM((RPW, H), jnp.float32),
            pltpu.SemaphoreType.DMA,
            pltpu.SemaphoreType.DMA,
            pltpu.SemaphoreType.DMA,
            pltpu.SemaphoreType.DMA,
        ],
    )
    def sc_kernel(ids_hbm, table_hbm, out_hbm, idx_v,
                  buf0, buf1, buf2, buf3, out_v, sem0, sem1, sem2, sem3):
        wid = lax.axis_index("s") * NC + lax.axis_index("c")
        base = wid * RPW
        pltpu.sync_copy(ids_hbm.at[pl.ds(base, RPW)], idx_v)

        bufs = (buf0, buf1, buf2, buf3)
        sems = (sem0, sem1, sem2, sem3)

        def issue(r, h, k):
            pltpu.async_copy(
                table_hbm.at[idx_v.at[r, pl.ds(h * HR, HR)]], bufs[k], sems[k])

        def wait(r, h, k):
            pltpu.make_async_copy(
                table_hbm.at[idx_v.at[r, pl.ds(h * HR, HR)]],
                bufs[k], sems[k]).wait()

        def reduce_half(buf, accs):
            def body(i, a):
                return tuple(a[j] + buf[i, pl.ds(_LANES * j, _LANES)]
                             for j in range(CH))
            return lax.fori_loop(0, HR, body, accs)

        def store(accs, r):
            for j in range(CH):
                out_v[r, pl.ds(_LANES * j, _LANES)] = accs[j]

        zeros = tuple(jnp.zeros((_LANES,), jnp.float32) for _ in range(CH))

        issue(0, 0, 0)
        issue(0, 1, 1)
        issue(1, 0, 2)
        issue(1, 1, 3)

        def loop_body(g, carry):
            r0 = 2 * g
            more = g < RPW // 2 - 1

            wait(r0, 0, 0)
            accs = reduce_half(buf0, zeros)

            @pl.when(more)
            def _():
                issue(r0 + 2, 0, 0)

            wait(r0, 1, 1)
            accs = reduce_half(buf1, accs)

            @pl.when(more)
            def _():
                issue(r0 + 2, 1, 1)

            store(accs, r0)

            wait(r0 + 1, 0, 2)
            accs = reduce_half(buf2, zeros)

            @pl.when(more)
            def _():
                issue(r0 + 3, 0, 2)

            wait(r0 + 1, 1, 3)
            accs = reduce_half(buf3, accs)

            @pl.when(more)
            def _():
                issue(r0 + 3, 1, 3)

            store(accs, r0 + 1)
            return carry

        lax.fori_loop(0, RPW // 2, loop_body, 0)
        pltpu.sync_copy(out_v, out_hbm.at[pl.ds(base, RPW)])

    return sc_kernel(ids, table)


def _tc_mlp(sums, pos, W1, b1, W2, b2, inv_l):
    B, H = sums.shape
    L = pos.shape[0]
    BM = 512

    def mlp_kernel(s_ref, pos_ref, w1_ref, b1_ref, w2_ref, b2_ref, o_ref):
        pos_mean = jnp.mean(pos_ref[...], axis=0, keepdims=True)
        x = s_ref[...] * inv_l + pos_mean
        h = lax.dot_general(x, w1_ref[...], (((1,), (1,)), ((), ())),
                            preferred_element_type=jnp.float32)
        h = jnp.maximum(h + b1_ref[...], 0.0)
        o = lax.dot_general(h, w2_ref[...], (((1,), (1,)), ((), ())),
                            preferred_element_type=jnp.float32)
        o_ref[...] = o + b2_ref[...]

    return pl.pallas_call(
        mlp_kernel,
        grid=(B // BM,),
        in_specs=[
            pl.BlockSpec((BM, H), lambda i: (i, 0)),
            pl.BlockSpec((L, H), lambda i: (0, 0)),
            pl.BlockSpec(W1.shape, lambda i: (0, 0)),
            pl.BlockSpec((1, H), lambda i: (0, 0)),
            pl.BlockSpec(W2.shape, lambda i: (0, 0)),
            pl.BlockSpec((1, H), lambda i: (0, 0)),
        ],
        out_specs=pl.BlockSpec((BM, W2.shape[0]), lambda i: (i, 0)),
        out_shape=jax.ShapeDtypeStruct((B, W2.shape[0]), jnp.float32),
    )(sums, pos, W1, b1, W2, b2)


def kernel(input_ids, emb_table, pos_table, W1, b1, W2, b2):
    ids = input_ids.astype(jnp.int32)
    L = ids.shape[1]
    sums = _sc_gather_sum(ids, emb_table)
    return _tc_mlp(sums, pos_table[:L], W1, b1.reshape(1, -1),
                   W2, b2.reshape(1, -1), 1.0 / L)
